# Initial kernel scaffold; baseline (speedup 1.0000x reference)
import jax
import jax.numpy as jnp
from jax.experimental import pallas as pl


def kernel(x, edge_index, W1_self, W1_neigh, b1, W2_self, W2_neigh, b2):
    raise NotImplementedError("write your pallas kernel here")


# speedup vs baseline: 81.6919x; 81.6919x over previous
import jax
import jax.numpy as jnp
from jax.experimental import pallas as pl


def _roll(h, shift, axis):
    return jnp.roll(h, shift, axis=axis)


def _body(x_ref, w1s_ref, w1n_ref, b1_ref, w2s_ref, w2n_ref, b2_ref, o_ref):
    h = x_ref[0]
    nx, ny, f = h.shape

    def sage(h3, ws, wn, b, f_out):
        agg = (_roll(h3, 1, 0) + _roll(h3, -1, 0)
               + _roll(h3, 1, 1) + _roll(h3, -1, 1)) * 0.25
        fin = h3.shape[-1]
        h2 = h3.reshape(nx * ny, fin)
        a2 = agg.reshape(nx * ny, fin)
        out = (jnp.dot(h2, ws, preferred_element_type=jnp.float32)
               + jnp.dot(a2, wn, preferred_element_type=jnp.float32)
               + b)
        return out.reshape(nx, ny, f_out)

    fh = w1s_ref.shape[1]
    h1 = jax.nn.relu(sage(h, w1s_ref[...], w1n_ref[...], b1_ref[...], fh))
    fo = w2s_ref.shape[1]
    h2 = jax.nn.relu(sage(h1, w2s_ref[...], w2n_ref[...], b2_ref[...], fo))
    o_ref[0] = h2


def kernel(x, edge_index, W1_self, W1_neigh, b1, W2_self, W2_neigh, b2):
    Bsz, T, nx, ny, F = x.shape
    FH = W1_self.shape[1]
    FO = W2_self.shape[1]
    xs = x.reshape(T, nx, ny, F)

    wspec = pl.BlockSpec((F, FH), lambda t: (0, 0))
    w2spec = pl.BlockSpec((FH, FO), lambda t: (0, 0))
    bspec = lambda f: pl.BlockSpec((1, f), lambda t: (0, 0))

    out = pl.pallas_call(
        _body,
        grid=(T,),
        in_specs=[
            pl.BlockSpec((1, nx, ny, F), lambda t: (t, 0, 0, 0)),
            wspec, wspec, bspec(FH),
            w2spec, w2spec, bspec(FO),
        ],
        out_specs=pl.BlockSpec((1, nx, ny, FO), lambda t: (t, 0, 0, 0)),
        out_shape=jax.ShapeDtypeStruct((T, nx, ny, FO), jnp.float32),
    )(xs, W1_self, W1_neigh, b1.reshape(1, FH),
      W2_self, W2_neigh, b2.reshape(1, FO))
    return out.reshape(Bsz, T, nx, ny, FO)
